# baseline (speedup 1.0000x reference)
import jax
import jax.numpy as jnp
from jax.experimental import pallas as pl
from jax.experimental.pallas import tpu as pltpu

_HALF = 6400


def _mlp_half(x, rbf, wrbf, wa, b1, w2, b2, wout, oshape):
    def silu(v):
        return 0.5 * v * (1.0 + jnp.tanh(0.5 * v))

    h = jnp.dot(rbf, wrbf, preferred_element_type=jnp.float32) * x
    z1t = jax.lax.dot_general(wa, h, (((0,), (1,)), ((), ())),
                              preferred_element_type=jnp.float32)
    ot = silu(z1t + b1)
    z2t = jax.lax.dot_general(w2, ot, (((0,), (0,)), ((), ())),
                              preferred_element_type=jnp.float32)
    ot = silu(z2t + b2)
    outt = jax.lax.dot_general(wout, ot, (((0,), (0,)), ((), ())),
                               preferred_element_type=jnp.float32)
    return outt.reshape(oshape)


def _mlp_block(xa_ref, xb_ref, rbfa_ref, rbfb_ref, wrbf_ref, wup_ref, w1_ref,
               b1_ref, w2_ref, b2_ref, wout_ref, oa_ref, ob_ref):
    wa = jnp.dot(wup_ref[...], w1_ref[...],
                 preferred_element_type=jnp.float32)
    args = (wrbf_ref[...], wa, b1_ref[...], w2_ref[...], b2_ref[...],
            wout_ref[...])
    oa_ref[...] = _mlp_half(xa_ref[...], rbfa_ref[...], *args, oa_ref.shape)
    ob_ref[...] = _mlp_half(xb_ref[...], rbfb_ref[...], *args, ob_ref.shape)


def kernel(x, rbf, i, num_nodes, W_rbf, W_up, W1, b1, W2, b2, W_out):
    del i, num_nodes
    E, H = x.shape
    R = rbf.shape[1]
    D = W_up.shape[1]
    b1 = b1.reshape(D, 1)
    b2 = b2.reshape(D, 1)

    n = E // (2 * _HALF)
    P = _HALF // 128
    rows_a = pl.BlockSpec((_HALF, H), lambda m: (2 * m, 0))
    rows_b = pl.BlockSpec((_HALF, H), lambda m: (2 * m + 1, 0))
    rbf_a = pl.BlockSpec((_HALF, R), lambda m: (2 * m, 0))
    rbf_b = pl.BlockSpec((_HALF, R), lambda m: (2 * m + 1, 0))
    rep_spec = lambda shape: pl.BlockSpec(shape, lambda m: (0, 0))
    out_spec = pl.BlockSpec((P, 1, 128), lambda m: (m, 0, 0))

    oa, ob = pl.pallas_call(
        _mlp_block,
        grid=(n,),
        in_specs=[
            rows_a, rows_b,
            rbf_a, rbf_b,
            rep_spec((R, H)),
            rep_spec((H, D)),
            rep_spec((D, D)),
            rep_spec((D, 1)),
            rep_spec((D, D)),
            rep_spec((D, 1)),
            rep_spec((D, 1)),
        ],
        out_specs=[out_spec, out_spec],
        out_shape=[jax.ShapeDtypeStruct((n * P, 1, 128), jnp.float32),
                   jax.ShapeDtypeStruct((n * P, 1, 128), jnp.float32)],
        compiler_params=pltpu.CompilerParams(
            dimension_semantics=("parallel",)),
    )(x, x, rbf, rbf, W_rbf, W_up, W1, b1, W2, b2, W_out)
    out = jnp.stack([oa.reshape(n, _HALF), ob.reshape(n, _HALF)], axis=1)
    return out.reshape(E, 1)
